# baseline (speedup 1.0000x reference)
import functools

import jax
import jax.numpy as jnp
from jax import lax
from jax.experimental import pallas as pl
from jax.experimental.pallas import tpu as pltpu
from jax.experimental.pallas import tpu_sc as plsc

_NC = 2
_NS = 16
_NW = _NC * _NS

_SEQ = 200
_D = 64
_B = 1024
_L = 16

_SGROUPS = 4
_BGROUPS = 8
_SPW = _SEQ // _SGROUPS
_BPW = _B // _BGROUPS


def _sc_embed(tbl2, pairs, idx, pos):
    mesh = plsc.VectorSubcoreMesh(core_axis_name="c", subcore_axis_name="s")

    @functools.partial(
        pl.kernel,
        mesh=mesh,
        out_type=jax.ShapeDtypeStruct((_SEQ, _D, _B), jnp.float32),
        compiler_params=pltpu.CompilerParams(
            use_tc_tiling_on_sc=False, needs_layout_passes=False),
        scratch_types=[
            pltpu.VMEM((_SPW, _BPW), jnp.int32),
            pltpu.VMEM((_SPW, _BPW), jnp.int32),
            pltpu.VMEM((_SEQ, _D), jnp.float32),
            pltpu.VMEM((_BPW, 128), jnp.float32),
            pltpu.VMEM((_BPW, 128), jnp.float32),
            pltpu.VMEM((_D, _BPW), jnp.float32),
            pltpu.VMEM((_D, _BPW), jnp.float32),
            pltpu.SemaphoreType.DMA,
            pltpu.SemaphoreType.DMA,
            pltpu.SemaphoreType.DMA,
            pltpu.SemaphoreType.DMA,
        ],
    )
    def k(tbl_hbm, pair_hbm, idx_hbm, pos_hbm, out_hbm,
          pair_v, idx_v, pos_v, row0, row1, ob0, ob1, sg0, sg1, so0, so1):
        wid = lax.axis_index("s") * _NC + lax.axis_index("c")
        s0 = (wid // _BGROUPS) * _SPW
        b0 = (wid % _BGROUPS) * _BPW

        pltpu.sync_copy(pair_hbm.at[wid], pair_v)
        pltpu.sync_copy(idx_hbm.at[wid], idx_v)
        pltpu.sync_copy(pos_hbm, pos_v)

        rows = (row0, row1)
        obs = (ob0, ob1)
        gsems = (sg0, sg1)
        osems = (so0, so1)
        iota = lax.iota(jnp.int32, _L)

        def gather(p, h, issue):
            c = (pltpu.async_copy if issue else pltpu.make_async_copy)
            return c(tbl_hbm.at[pair_v.at[p]], rows[h], gsems[h])

        def writeback(p, h, issue):
            c = (pltpu.async_copy if issue else pltpu.make_async_copy)
            return c(obs[h], out_hbm.at[s0 + p].at[:, pl.ds(b0, _BPW)],
                     osems[h])

        def compute(p, h):
            rbuf = rows[h]
            ob = obs[h]
            s_glob = s0 + p
            pars = [(idx_v[p, pl.ds(rb * _L, _L)] & 1) * _D for rb in range(8)]
            rids = [iota + rb * _L for rb in range(8)]

            @plsc.parallel_loop(0, _D, unroll=2)
            def body(j):
                ps = plsc.load_gather(
                    pos_v, [jnp.full((_L,), s_glob, jnp.int32),
                            jnp.full((_L,), j, jnp.int32)])
                for rb in range(8):
                    v = plsc.load_gather(rbuf, [rids[rb], pars[rb] + j])
                    ob[j, pl.ds(rb * _L, _L)] = v * 8.0 + ps

        gather(0, 0, True)
        gather(1, 1, True)

        def step(i, _):
            for h in range(2):
                p = 2 * i + h
                gather(p, h, False).wait()
                compute(p, h)

                @pl.when(p >= 2)
                def _():
                    writeback(p - 2, h, False).wait()

                writeback(p, h, True)

                @pl.when(p + 2 < _SPW)
                def _():
                    gather(p + 2, h, True)
            return 0

        lax.fori_loop(0, _SPW // 2, step, 0)
        writeback(_SPW - 2, 0, False).wait()
        writeback(_SPW - 1, 1, False).wait()

    return k(tbl2, pairs, idx, pos)


def kernel(inp, table, pos_encoding):
    batch, seq = inp.shape
    d = table.shape[1]
    tbl2 = jnp.reshape(table, (table.shape[0] // 2, 2 * d))
    it = (inp.T.reshape(_SGROUPS, _SPW, _BGROUPS, _BPW)
          .transpose(0, 2, 1, 3).reshape(_NW, _SPW, _BPW))
    pairs = it >> 1
    pos = pos_encoding[0, :seq, :]
    out_t = _sc_embed(tbl2, pairs, it, pos)
    return out_t.transpose(2, 0, 1)


# speedup vs baseline: 1.4964x; 1.4964x over previous
import functools

import jax
import jax.numpy as jnp
from jax import lax
from jax.experimental import pallas as pl
from jax.experimental.pallas import tpu as pltpu
from jax.experimental.pallas import tpu_sc as plsc

_NC = 2
_NS = 16
_NW = _NC * _NS

_SEQ = 200
_D = 64
_B = 1024
_L = 16

_SGROUPS = 4
_BGROUPS = 8
_SPW = _SEQ // _SGROUPS
_BPW = _B // _BGROUPS


_VB = 2048


def _tc_repack(tt):
    v = tt.shape[1]
    grid = (v // 2 + _VB - 1) // _VB
    nin = (v + _VB - 1) // _VB

    def body(a_ref, b_ref, out_ref):
        out_ref[...] = jnp.concatenate(
            [a_ref[...].T, b_ref[...].T], axis=1)

    return pl.pallas_call(
        body,
        grid=(grid,),
        in_specs=[
            pl.BlockSpec((_D, _VB), lambda g: (0, jnp.minimum(2 * g, nin - 1))),
            pl.BlockSpec((_D, _VB),
                         lambda g: (0, jnp.minimum(2 * g + 1, nin - 1))),
        ],
        out_specs=pl.BlockSpec((_VB, 2 * _D), lambda g: (g, 0)),
        out_shape=jax.ShapeDtypeStruct((grid * _VB, 2 * _D), jnp.float32),
    )(tt, tt)


def _sc_embed(tbl2, pairs, idx, pos):
    mesh = plsc.VectorSubcoreMesh(core_axis_name="c", subcore_axis_name="s")

    @functools.partial(
        pl.kernel,
        mesh=mesh,
        out_type=jax.ShapeDtypeStruct((_SEQ, _D, _B), jnp.float32),
        compiler_params=pltpu.CompilerParams(
            use_tc_tiling_on_sc=False, needs_layout_passes=False),
        scratch_types=[
            pltpu.VMEM((_SPW, _BPW), jnp.int32),
            pltpu.VMEM((_SPW, _BPW), jnp.int32),
            pltpu.VMEM((_SEQ, _D), jnp.float32),
            pltpu.VMEM((_BPW, 128), jnp.float32),
            pltpu.VMEM((_BPW, 128), jnp.float32),
            pltpu.VMEM((_D, _BPW), jnp.float32),
            pltpu.VMEM((_D, _BPW), jnp.float32),
            pltpu.SemaphoreType.DMA,
            pltpu.SemaphoreType.DMA,
            pltpu.SemaphoreType.DMA,
            pltpu.SemaphoreType.DMA,
        ],
    )
    def k(tbl_hbm, pair_hbm, idx_hbm, pos_hbm, out_hbm,
          pair_v, idx_v, pos_v, row0, row1, ob0, ob1, sg0, sg1, so0, so1):
        wid = lax.axis_index("s") * _NC + lax.axis_index("c")
        s0 = (wid // _BGROUPS) * _SPW
        b0 = (wid % _BGROUPS) * _BPW

        pltpu.sync_copy(pair_hbm.at[wid], pair_v)
        pltpu.sync_copy(idx_hbm.at[wid], idx_v)
        pltpu.sync_copy(pos_hbm, pos_v)

        rows = (row0, row1)
        obs = (ob0, ob1)
        gsems = (sg0, sg1)
        osems = (so0, so1)
        iota = lax.iota(jnp.int32, _L)

        def gather(p, h, issue):
            c = (pltpu.async_copy if issue else pltpu.make_async_copy)
            return c(tbl_hbm.at[pair_v.at[p]], rows[h], gsems[h])

        def writeback(p, h, issue):
            c = (pltpu.async_copy if issue else pltpu.make_async_copy)
            return c(obs[h], out_hbm.at[s0 + p].at[:, pl.ds(b0, _BPW)],
                     osems[h])

        def compute(p, h):
            rbuf = rows[h]
            ob = obs[h]
            s_glob = s0 + p
            pars = [((idx_v[p, pl.ds(rb * _L, _L)] >> 11) & 1) * _D
                    for rb in range(8)]
            rids = [iota + rb * _L for rb in range(8)]

            @plsc.parallel_loop(0, _D, unroll=2)
            def body(j):
                ps = plsc.load_gather(
                    pos_v, [jnp.full((_L,), s_glob, jnp.int32),
                            jnp.full((_L,), j, jnp.int32)])
                for rb in range(8):
                    v = plsc.load_gather(rbuf, [rids[rb], pars[rb] + j])
                    ob[j, pl.ds(rb * _L, _L)] = v * 8.0 + ps

        gather(0, 0, True)
        gather(1, 1, True)

        def step(i, _):
            for h in range(2):
                p = 2 * i + h
                gather(p, h, False).wait()
                compute(p, h)

                @pl.when(p >= 2)
                def _():
                    writeback(p - 2, h, False).wait()

                writeback(p, h, True)

                @pl.when(p + 2 < _SPW)
                def _():
                    gather(p + 2, h, True)
            return 0

        lax.fori_loop(0, _SPW // 2, step, 0)
        writeback(_SPW - 2, 0, False).wait()
        writeback(_SPW - 1, 1, False).wait()

    return k(tbl2, pairs, idx, pos)


def kernel(inp, table, pos_encoding):
    batch, seq = inp.shape
    d = table.shape[1]
    tbl2 = _tc_repack(table.T)
    it = (inp.T.reshape(_SGROUPS, _SPW, _BGROUPS, _BPW)
          .transpose(0, 2, 1, 3).reshape(_NW, _SPW, _BPW))
    pairs = (it >> 12) * _VB + (it & (_VB - 1))
    pos = pos_encoding[0, :seq, :]
    out_t = _sc_embed(tbl2, pairs, it, pos)
    return out_t.transpose(2, 0, 1)


# speedup vs baseline: 1.8454x; 1.2332x over previous
import functools

import jax
import jax.numpy as jnp
from jax import lax
from jax.experimental import pallas as pl
from jax.experimental.pallas import tpu as pltpu
from jax.experimental.pallas import tpu_sc as plsc

_NC = 2
_NS = 16
_NW = _NC * _NS

_SEQ = 200
_D = 64
_B = 1024
_L = 16

_SGROUPS = 4
_BGROUPS = 8
_SPW = _SEQ // _SGROUPS
_BPW = _B // _BGROUPS

_VB = 8192
_VSH = 13


def _tc_repack(tt):
    v = tt.shape[1]
    grid = (v // 2 + _VB - 1) // _VB
    nin = (v + _VB - 1) // _VB

    def body(a_ref, b_ref, out_ref):
        out_ref[...] = jnp.concatenate(
            [a_ref[...].T, b_ref[...].T], axis=1)

    return pl.pallas_call(
        body,
        grid=(grid,),
        in_specs=[
            pl.BlockSpec((_D, _VB), lambda g: (0, jnp.minimum(2 * g, nin - 1))),
            pl.BlockSpec((_D, _VB),
                         lambda g: (0, jnp.minimum(2 * g + 1, nin - 1))),
        ],
        out_specs=pl.BlockSpec((_VB, 2 * _D), lambda g: (g, 0)),
        out_shape=jax.ShapeDtypeStruct((grid * _VB, 2 * _D), jnp.float32),
    )(tt, tt)


def _sc_embed(tbl2, gidx, pos):
    mesh = plsc.VectorSubcoreMesh(core_axis_name="c", subcore_axis_name="s")

    @functools.partial(
        pl.kernel,
        mesh=mesh,
        out_type=jax.ShapeDtypeStruct((_SEQ, _D, _B), jnp.float32),
        compiler_params=pltpu.CompilerParams(
            use_tc_tiling_on_sc=False, needs_layout_passes=False),
        scratch_types=[
            pltpu.VMEM((_SPW, _BPW), jnp.int32),
            pltpu.VMEM((_SEQ, _D), jnp.float32),
            pltpu.VMEM((_BPW, _D), jnp.float32),
            pltpu.VMEM((_BPW, _D), jnp.float32),
            pltpu.VMEM((_D, _BPW), jnp.float32),
            pltpu.VMEM((_D, _BPW), jnp.float32),
            pltpu.SemaphoreType.DMA,
            pltpu.SemaphoreType.DMA,
            pltpu.SemaphoreType.DMA,
            pltpu.SemaphoreType.DMA,
        ],
    )
    def k(tbl_hbm, gidx_hbm, pos_hbm, out_hbm,
          gidx_v, pos_v, row0, row1, ob0, ob1, sg0, sg1, so0, so1):
        wid = lax.axis_index("s") * _NC + lax.axis_index("c")
        s0 = (wid // _BGROUPS) * _SPW
        b0 = (wid % _BGROUPS) * _BPW

        pltpu.sync_copy(gidx_hbm.at[wid], gidx_v)
        pltpu.sync_copy(pos_hbm, pos_v)

        rows = (row0, row1)
        obs = (ob0, ob1)
        gsems = (sg0, sg1)
        osems = (so0, so1)
        iota = lax.iota(jnp.int32, _L)

        def gather(p, h, issue):
            c = (pltpu.async_copy if issue else pltpu.make_async_copy)
            return c(tbl_hbm.at[gidx_v.at[p]], rows[h], gsems[h])

        def writeback(p, h, issue):
            c = (pltpu.async_copy if issue else pltpu.make_async_copy)
            return c(obs[h], out_hbm.at[s0 + p].at[:, pl.ds(b0, _BPW)],
                     osems[h])

        def compute(p, h):
            rbuf = rows[h]
            ob = obs[h]
            s_glob = s0 + p
            rids = [iota + rb * _L for rb in range(8)]

            @plsc.parallel_loop(0, _D, unroll=2)
            def body(j):
                jv = jnp.full((_L,), j, jnp.int32)
                ps = plsc.load_gather(
                    pos_v, [jnp.full((_L,), s_glob, jnp.int32), jv])
                for rb in range(8):
                    v = plsc.load_gather(rbuf, [rids[rb], jv])
                    ob[j, pl.ds(rb * _L, _L)] = v * 8.0 + ps

        gather(0, 0, True)
        gather(1, 1, True)

        def step(i, _):
            for h in range(2):
                p = 2 * i + h
                gather(p, h, False).wait()
                compute(p, h)

                @pl.when(p >= 2)
                def _():
                    writeback(p - 2, h, False).wait()

                writeback(p, h, True)

                @pl.when(p + 2 < _SPW)
                def _():
                    gather(p + 2, h, True)
            return 0

        lax.fori_loop(0, _SPW // 2, step, 0)
        writeback(_SPW - 2, 0, False).wait()
        writeback(_SPW - 1, 1, False).wait()

    return k(tbl2, gidx, pos)


def kernel(inp, table, pos_encoding):
    batch, seq = inp.shape
    d = table.shape[1]
    tbl2 = _tc_repack(table.T)
    tbl2 = tbl2.reshape(tbl2.shape[0] * 2, d)
    it = (inp.T.reshape(_SGROUPS, _SPW, _BGROUPS, _BPW)
          .transpose(0, 2, 1, 3).reshape(_NW, _SPW, _BPW))
    gidx = ((it >> (_VSH + 1)) * (2 * _VB) + ((it & (_VB - 1)) << 1)
            + ((it >> _VSH) & 1))
    pos = pos_encoding[0, :seq, :]
    out_t = _sc_embed(tbl2, gidx, pos)
    return out_t.transpose(2, 0, 1)
